# baseline (speedup 1.0000x reference)
import jax
import jax.numpy as jnp
from jax.experimental import pallas as pl
from jax.experimental.pallas import tpu as pltpu

N = 4096
IN_CH = 512
HID = 64
BLOCK = 512


def _body(seq_ref, adj_ref, wt_ref, b_ref, a_ref, out_ref, fts_ref):
    i = pl.program_id(0)

    @pl.when(i == 0)
    def _():
        fts_ref[...] = jnp.dot(
            seq_ref[...], wt_ref[...], preferred_element_type=jnp.float32
        )

    out = jnp.dot(adj_ref[...], fts_ref[...], preferred_element_type=jnp.float32)
    out = out + b_ref[...]
    a = a_ref[0, 0]
    out_ref[...] = jnp.where(out > 0.0, out, a * out)


def kernel(seq, adj, W, bias, prelu_a):
    wt = W.T
    b2 = bias.reshape(1, HID)
    a2 = jnp.asarray(prelu_a, jnp.float32).reshape(1, 1)

    grid = (N // BLOCK,)
    return pl.pallas_call(
        _body,
        grid=grid,
        in_specs=[
            pl.BlockSpec((N, IN_CH), lambda i: (0, 0)),
            pl.BlockSpec((BLOCK, N), lambda i: (i, 0)),
            pl.BlockSpec((IN_CH, HID), lambda i: (0, 0)),
            pl.BlockSpec((1, HID), lambda i: (0, 0)),
            pl.BlockSpec(memory_space=pltpu.SMEM),
        ],
        out_specs=pl.BlockSpec((BLOCK, HID), lambda i: (i, 0)),
        out_shape=jax.ShapeDtypeStruct((N, HID), jnp.float32),
        scratch_shapes=[pltpu.VMEM((N, HID), jnp.float32)],
    )(seq, adj, wt, b2, a2)
